# baseline (speedup 1.0000x reference)
import functools

import jax
import jax.numpy as jnp
from jax import lax
from jax.experimental import pallas as pl
from jax.experimental.pallas import tpu as pltpu
from jax.experimental.pallas import tpu_sc as plsc

_N = 10000
_E = 320000
_D = 128

_NCORES = 2
_NSUB = 16
_CHUNK = 40
_BN = 1000

_RS = 96
_RD = 16
_RA = 80


def _make_edge_pass(per_head):
    e_per_sc = _E // _NCORES
    e_per_tile = e_per_sc // _NSUB
    n_chunks = e_per_tile // _CHUNK
    rows_u = 624
    tail0 = rows_u * _NSUB
    tail_rows = _N - tail0
    zrows = 48

    mesh = plsc.VectorSubcoreMesh(core_axis_name="c", subcore_axis_name="s",
                                  num_cores=_NCORES, num_subcores=_NSUB)

    @functools.partial(
        pl.kernel,
        out_type=jax.ShapeDtypeStruct((_NCORES, _N, _RA), jnp.float32),
        mesh=mesh,
        compiler_params=pltpu.CompilerParams(use_tc_tiling_on_sc=False,
                                             needs_layout_passes=False),
        scratch_types=[
            pltpu.VMEM((n_chunks, _CHUNK), jnp.int32),
            pltpu.VMEM((n_chunks, _CHUNK), jnp.int32),
        ] + [pltpu.VMEM((_CHUNK, _RS), jnp.bfloat16)] * 8
          + [pltpu.VMEM((_CHUNK, _RD), jnp.float32)] * 8
          + [pltpu.VMEM((_CHUNK, _RA), jnp.float32)] * 8
          + [pltpu.VMEM((zrows, _RA), jnp.float32),
             pltpu.VMEM((4, 16), jnp.int32),
             pltpu.VMEM_SHARED((_N, _RA), jnp.float32)]
          + [pltpu.SemaphoreType.DMA] * 16,
    )
    def edge_kernel(src_h, dst_h, bysrc_h, bydst_h, lanes_h, out_h,
                    src_i, dst_i, *rest):
        rsrc = rest[0:8]
        rdst = rest[8:16]
        msg = rest[16:24]
        zbuf, lanes_v, acc = rest[24:27]
        gsem = rest[27:35]
        ssem = rest[35:43]
        cid = lax.axis_index("c")
        sid = lax.axis_index("s")
        tid = cid * _NSUB + sid
        row0 = sid * rows_u

        idx_copies = [
            pltpu.async_copy(src_h.at[tid], src_i, gsem[0]),
            pltpu.async_copy(dst_h.at[tid], dst_i, gsem[1]),
        ]
        pltpu.sync_copy(lanes_h, lanes_v)

        zero16 = jnp.zeros((16,), jnp.float32)

        def zero_row(i, _):
            for j in range(_RA // 16):
                zbuf[i, pl.ds(16 * j, 16)] = zero16
            return 0

        def issue(c, b):
            pltpu.async_copy(bysrc_h.at[src_i.at[c]], rsrc[b], gsem[b])
            pltpu.async_copy(bydst_h.at[dst_i.at[c]], rdst[b], gsem[b])

        def wait_gathers(b):
            pltpu.make_async_copy(bysrc_h.at[src_i.at[0]], rsrc[b],
                                  gsem[b]).wait()
            pltpu.make_async_copy(bydst_h.at[dst_i.at[0]], rdst[b],
                                  gsem[b]).wait()

        def wait_scatter(b):
            pltpu.make_async_copy(msg[b], acc.at[dst_i.at[0]],
                                  ssem[b]).wait()

        unroll = 8

        def edge_body(b):
            lanes = [lanes_v[j, :] for j in range(4)]

            def one_edge(ei):
                s16, _ = plsc.unpack(rsrc[b][ei, pl.ds(64, 32)],
                                     format=plsc.PackFormat.INTERLEAVED,
                                     preferred_element_type=jnp.float32)
                a = s16 + rdst[b][ei, pl.ds(0, 16)]
                w = jnp.exp(jnp.maximum(a, 0.2 * a))
                msg[b][ei, pl.ds(64, 16)] = w
                for g in range(2):
                    h0, h1 = plsc.unpack(rsrc[b][ei, pl.ds(32 * g, 32)],
                                         format=plsc.PackFormat.INTERLEAVED,
                                         preferred_element_type=jnp.float32)
                    for jj, hv in ((2 * g, h0), (2 * g + 1, h1)):
                        if per_head:
                            wj = w.at[lanes[jj]].get(
                                mode="promise_in_bounds")
                        else:
                            wj = w
                        msg[b][ei, pl.ds(16 * jj, 16)] = hv * wj

            plsc.parallel_loop(0, _CHUNK, 1, unroll=unroll)(one_edge)

        lax.fori_loop(0, zrows, zero_row, 0)
        for c in idx_copies:
            c.wait()
        for b in range(8):
            issue(b, b)
        for k in range(rows_u // zrows):
            pltpu.sync_copy(zbuf, acc.at[pl.ds(row0 + k * zrows, zrows)])

        @pl.when(sid == _NSUB - 1)
        def _():
            pltpu.sync_copy(zbuf.at[pl.ds(0, tail_rows)],
                            acc.at[pl.ds(tail0, tail_rows)])

        plsc.subcore_barrier()

        def stage(c, b):
            wait_gathers(b)

            @pl.when(c >= 8)
            def _():
                wait_scatter(b)
            edge_body(b)
            pltpu.async_copy(msg[b], acc.at[dst_i.at[c]], ssem[b], add=True)

            @pl.when(c + 8 < n_chunks)
            def _():
                issue(c + 8, b)

        def ring_body(k, _):
            for u in range(8):
                stage(8 * k + u, u)
            return 0

        lax.fori_loop(0, n_chunks // 8, ring_body, 0)
        stage(n_chunks - 2, 0)
        stage(n_chunks - 1, 1)
        for b in (2, 3, 4, 5, 6, 7, 0, 1):
            wait_scatter(b)
        plsc.subcore_barrier()
        pltpu.sync_copy(acc.at[pl.ds(row0, rows_u)],
                        out_h.at[cid, pl.ds(row0, rows_u)])

        @pl.when(sid == _NSUB - 1)
        def _():
            pltpu.sync_copy(acc.at[pl.ds(tail0, tail_rows)],
                            out_h.at[cid, pl.ds(tail0, tail_rows)])

    return edge_kernel


def _prep1(x, A1, B1):
    def body(x_ref, a_ref, b_ref, o1_ref, o2_ref):
        xv = x_ref[...]
        o1_ref[...] = jnp.dot(
            xv, a_ref[...],
            preferred_element_type=jnp.float32).astype(jnp.bfloat16)
        o2_ref[...] = jnp.dot(xv, b_ref[...], preferred_element_type=jnp.float32)

    return pl.pallas_call(
        body,
        grid=(_N // _BN,),
        in_specs=[
            pl.BlockSpec((_BN, _D), lambda i: (i, 0)),
            pl.BlockSpec((_D, _RS), lambda i: (0, 0)),
            pl.BlockSpec((_D, _RD), lambda i: (0, 0)),
        ],
        out_specs=[
            pl.BlockSpec((_BN, _RS), lambda i: (i, 0)),
            pl.BlockSpec((_BN, _RD), lambda i: (i, 0)),
        ],
        out_shape=[
            jax.ShapeDtypeStruct((_N, _RS), jnp.bfloat16),
            jax.ShapeDtypeStruct((_N, _RD), jnp.float32),
        ],
    )(x, A1, B1)


def _finalize1_prep2(acc1, bysrc1, bydst1, b1, E16, Pinv, A2, B2):
    def body(acc_ref, bs_ref, bd_ref, b1_ref, e_ref, p_ref, a2_ref, b2_ref,
             o1_ref, o2_ref):
        num = acc_ref[0, :, 0:64] + acc_ref[1, :, 0:64]
        den16 = acc_ref[0, :, 64:80] + acc_ref[1, :, 64:80]
        ev = e_ref[...]
        den = jnp.dot(den16, ev, preferred_element_type=jnp.float32)
        bs = jnp.dot(bs_ref[...].astype(jnp.float32), p_ref[...],
                     preferred_element_type=jnp.float32)
        h = bs[:, 0:64]
        a0 = bs[:, 64:80] + bd_ref[...]
        ws16 = jnp.exp(jnp.maximum(a0, 0.2 * a0))
        ws = jnp.dot(ws16, ev, preferred_element_type=jnp.float32)
        o = (num + ws * h) / (den + ws + 1e-16) + b1_ref[...]
        h1 = jnp.where(o > 0, o, jnp.exp(o) - 1.0)
        o1_ref[...] = jnp.dot(
            h1, a2_ref[...],
            preferred_element_type=jnp.float32).astype(jnp.bfloat16)
        o2_ref[...] = jnp.dot(h1, b2_ref[...], preferred_element_type=jnp.float32)

    return pl.pallas_call(
        body,
        grid=(_N // _BN,),
        in_specs=[
            pl.BlockSpec((_NCORES, _BN, _RA), lambda i: (0, i, 0)),
            pl.BlockSpec((_BN, _RS), lambda i: (i, 0)),
            pl.BlockSpec((_BN, _RD), lambda i: (i, 0)),
            pl.BlockSpec((1, 64), lambda i: (0, 0)),
            pl.BlockSpec((16, 64), lambda i: (0, 0)),
            pl.BlockSpec((_RS, _RS), lambda i: (0, 0)),
            pl.BlockSpec((64, _RS), lambda i: (0, 0)),
            pl.BlockSpec((64, _RD), lambda i: (0, 0)),
        ],
        out_specs=[
            pl.BlockSpec((_BN, _RS), lambda i: (i, 0)),
            pl.BlockSpec((_BN, _RD), lambda i: (i, 0)),
        ],
        out_shape=[
            jax.ShapeDtypeStruct((_N, _RS), jnp.bfloat16),
            jax.ShapeDtypeStruct((_N, _RD), jnp.float32),
        ],
    )(acc1, bysrc1, bydst1, b1, E16, Pinv, A2, B2)


def _finalize2(acc2, bysrc2, bydst2, b2, Pinv):
    def body(acc_ref, bs_ref, bd_ref, b2_ref, p_ref, o_ref):
        num = acc_ref[0, :, 0:64] + acc_ref[1, :, 0:64]
        den = acc_ref[0, :, 64:65] + acc_ref[1, :, 64:65]
        bs = jnp.dot(bs_ref[...].astype(jnp.float32), p_ref[...],
                     preferred_element_type=jnp.float32)
        h = bs[:, 0:64]
        a0 = bs[:, 64:65] + bd_ref[:, 0:1]
        ws = jnp.exp(jnp.maximum(a0, 0.2 * a0))
        o = (num + ws * h) / (den + ws + 1e-16) + b2_ref[...]
        m = jnp.max(o, axis=1, keepdims=True)
        z = o - m
        o_ref[...] = z - jnp.log(jnp.sum(jnp.exp(z), axis=1, keepdims=True))

    return pl.pallas_call(
        body,
        grid=(_N // _BN,),
        in_specs=[
            pl.BlockSpec((_NCORES, _BN, _RA), lambda i: (0, i, 0)),
            pl.BlockSpec((_BN, _RS), lambda i: (i, 0)),
            pl.BlockSpec((_BN, _RD), lambda i: (i, 0)),
            pl.BlockSpec((1, 64), lambda i: (0, 0)),
            pl.BlockSpec((_RS, _RS), lambda i: (0, 0)),
        ],
        out_specs=pl.BlockSpec((_BN, 64), lambda i: (i, 0)),
        out_shape=jax.ShapeDtypeStruct((_N, 64), jnp.float32),
    )(acc2, bysrc2, bydst2, b2, Pinv)


def _head_mat(a):
    hh, cc = a.shape
    t = a[:, :, None] * jnp.eye(hh, dtype=a.dtype)[:, None, :]
    return t.reshape(hh * cc, hh)


_edge_pass_cache = {}


def _edge_pass(per_head):
    if per_head not in _edge_pass_cache:
        _edge_pass_cache[per_head] = _make_edge_pass(per_head)
    return _edge_pass_cache[per_head]


def kernel(x, edge_index, W1, a_src1, a_dst1, b1, W2, a_src2, a_dst2, b2):
    n_tiles = _NCORES * _NSUB
    n_chunks = _E // n_tiles // _CHUNK
    src = edge_index[0].reshape(n_tiles, n_chunks, _CHUNK)
    dst = edge_index[1].reshape(n_tiles, n_chunks, _CHUNK)

    f32 = W1.dtype
    zpad = jnp.zeros((64, 8), f32)
    lperm = []
    for g in range(3):
        for k in range(16):
            lperm.extend([32 * g + k, 32 * g + 16 + k])
    lperm = jnp.asarray(lperm, dtype=jnp.int32)
    Pinv = jnp.zeros((_RS, _RS), f32).at[jnp.arange(_RS), lperm].set(1.0)

    A1log = W1 @ jnp.concatenate([jnp.eye(64, dtype=f32), _head_mat(a_src1),
                                  jnp.zeros((64, 24), f32)], axis=1)
    A1 = A1log[:, lperm]
    B1 = W1 @ jnp.concatenate([_head_mat(a_dst1), zpad], axis=1)
    E16 = jnp.concatenate(
        [jnp.repeat(jnp.eye(8, dtype=f32), 8, axis=1), jnp.zeros((8, 64), f32)],
        axis=0)
    s2 = W2 @ a_src2.T
    d2 = W2 @ a_dst2.T
    A2log = jnp.concatenate([W2, jnp.broadcast_to(s2, (64, 16)),
                             jnp.zeros((64, 16), f32)], axis=1)
    A2 = A2log[:, lperm]
    B2 = jnp.broadcast_to(d2, (64, 16))

    lanes = jnp.asarray(
        [[2 * j] * 8 + [2 * j + 1] * 8 for j in range(4)], dtype=jnp.int32)

    bysrc1, bydst1 = _prep1(x, A1, B1)
    acc1 = _edge_pass(True)(src, dst, bysrc1, bydst1, lanes)
    bysrc2, bydst2 = _finalize1_prep2(acc1, bysrc1, bydst1,
                                      b1.reshape(1, 64), E16, Pinv, A2, B2)
    acc2 = _edge_pass(False)(src, dst, bysrc2, bydst2, lanes)
    return _finalize2(acc2, bysrc2, bydst2, b2.reshape(1, 64), Pinv)


# speedup vs baseline: 1.1201x; 1.1201x over previous
import functools

import jax
import jax.numpy as jnp
from jax import lax
from jax.experimental import pallas as pl
from jax.experimental.pallas import tpu as pltpu
from jax.experimental.pallas import tpu_sc as plsc

_N = 10000
_E = 320000
_D = 128

_NCORES = 2
_NSUB = 16
_CHUNK = 80
_BN = 1000
_NBUF = 5

_RS = 96
_RD = 16
_RA = 80


def _make_edge_pass(per_head):
    e_per_sc = _E // _NCORES
    e_per_tile = e_per_sc // _NSUB
    n_chunks = e_per_tile // _CHUNK
    rows_u = 624
    tail0 = rows_u * _NSUB
    tail_rows = _N - tail0
    zrows = 48

    mesh = plsc.VectorSubcoreMesh(core_axis_name="c", subcore_axis_name="s",
                                  num_cores=_NCORES, num_subcores=_NSUB)

    @functools.partial(
        pl.kernel,
        out_type=jax.ShapeDtypeStruct((_NCORES, _N, _RA), jnp.float32),
        mesh=mesh,
        compiler_params=pltpu.CompilerParams(use_tc_tiling_on_sc=False,
                                             needs_layout_passes=False),
        scratch_types=[
            pltpu.VMEM((n_chunks, _CHUNK), jnp.int32),
            pltpu.VMEM((n_chunks, _CHUNK), jnp.int32),
        ] + [pltpu.VMEM((_CHUNK, _RS), jnp.bfloat16)] * _NBUF
          + [pltpu.VMEM((_CHUNK, _RD), jnp.float32)] * _NBUF
          + [pltpu.VMEM((_CHUNK, _RA), jnp.float32)] * _NBUF
          + [pltpu.VMEM((4, 16), jnp.int32),
             pltpu.VMEM_SHARED((_N, _RA), jnp.float32)]
          + [pltpu.SemaphoreType.DMA] * (2 * _NBUF),
    )
    def edge_kernel(src_h, dst_h, bysrc_h, bydst_h, lanes_h, out_h,
                    src_i, dst_i, *rest):
        rsrc = rest[0:_NBUF]
        rdst = rest[_NBUF:2 * _NBUF]
        msg = rest[2 * _NBUF:3 * _NBUF]
        lanes_v, acc = rest[3 * _NBUF:3 * _NBUF + 2]
        gsem = rest[3 * _NBUF + 2:4 * _NBUF + 2]
        ssem = rest[4 * _NBUF + 2:5 * _NBUF + 2]
        cid = lax.axis_index("c")
        sid = lax.axis_index("s")
        tid = cid * _NSUB + sid
        row0 = sid * rows_u

        idx_copies = [
            pltpu.async_copy(src_h.at[tid], src_i, gsem[0]),
            pltpu.async_copy(dst_h.at[tid], dst_i, gsem[1]),
        ]
        pltpu.sync_copy(lanes_h, lanes_v)

        zero16 = jnp.zeros((16,), jnp.float32)
        zbuf = msg[0]

        def zero_row(i, _):
            for j in range(_RA // 16):
                zbuf[i, pl.ds(16 * j, 16)] = zero16
            return 0

        def issue(c, b):
            pltpu.async_copy(bysrc_h.at[src_i.at[c]], rsrc[b], gsem[b])
            pltpu.async_copy(bydst_h.at[dst_i.at[c]], rdst[b], gsem[b])

        def wait_gathers(b):
            pltpu.make_async_copy(bysrc_h.at[src_i.at[0]], rsrc[b],
                                  gsem[b]).wait()
            pltpu.make_async_copy(bydst_h.at[dst_i.at[0]], rdst[b],
                                  gsem[b]).wait()

        def wait_scatter(b):
            pltpu.make_async_copy(msg[b], acc.at[dst_i.at[0]],
                                  ssem[b]).wait()

        unroll = 8

        def edge_body(b):
            lanes = [lanes_v[j, :] for j in range(4)]

            def one_edge(ei):
                s16, _ = plsc.unpack(rsrc[b][ei, pl.ds(64, 32)],
                                     format=plsc.PackFormat.INTERLEAVED,
                                     preferred_element_type=jnp.float32)
                a = s16 + rdst[b][ei, pl.ds(0, 16)]
                w = jnp.exp(jnp.maximum(a, 0.2 * a))
                msg[b][ei, pl.ds(64, 16)] = w
                for g in range(2):
                    h0, h1 = plsc.unpack(rsrc[b][ei, pl.ds(32 * g, 32)],
                                         format=plsc.PackFormat.INTERLEAVED,
                                         preferred_element_type=jnp.float32)
                    for jj, hv in ((2 * g, h0), (2 * g + 1, h1)):
                        if per_head:
                            wj = w.at[lanes[jj]].get(
                                mode="promise_in_bounds")
                        else:
                            wj = w
                        msg[b][ei, pl.ds(16 * jj, 16)] = hv * wj

            plsc.parallel_loop(0, _CHUNK, 1, unroll=unroll)(one_edge)

        lax.fori_loop(0, _CHUNK, zero_row, 0)
        for c in idx_copies:
            c.wait()
        for b in range(_NBUF):
            issue(b, b)
        for k in range(rows_u // _CHUNK):
            pltpu.sync_copy(zbuf, acc.at[pl.ds(row0 + k * _CHUNK, _CHUNK)])
        rem = rows_u - (rows_u // _CHUNK) * _CHUNK
        pltpu.sync_copy(zbuf.at[pl.ds(0, rem)],
                        acc.at[pl.ds(row0 + rows_u - rem, rem)])

        @pl.when(sid == _NSUB - 1)
        def _():
            pltpu.sync_copy(zbuf.at[pl.ds(0, tail_rows)],
                            acc.at[pl.ds(tail0, tail_rows)])

        plsc.subcore_barrier()

        def stage(c, b):
            wait_gathers(b)

            @pl.when(c >= _NBUF)
            def _():
                wait_scatter(b)
            edge_body(b)
            pltpu.async_copy(msg[b], acc.at[dst_i.at[c]], ssem[b], add=True)

            @pl.when(c + _NBUF < n_chunks)
            def _():
                issue(c + _NBUF, b)

        def ring_body(k, _):
            for u in range(_NBUF):
                stage(_NBUF * k + u, u)
            return 0

        lax.fori_loop(0, n_chunks // _NBUF, ring_body, 0)
        for b in range(_NBUF):
            wait_scatter(b)
        plsc.subcore_barrier()
        pltpu.sync_copy(acc.at[pl.ds(row0, rows_u)],
                        out_h.at[cid, pl.ds(row0, rows_u)])

        @pl.when(sid == _NSUB - 1)
        def _():
            pltpu.sync_copy(acc.at[pl.ds(tail0, tail_rows)],
                            out_h.at[cid, pl.ds(tail0, tail_rows)])

    return edge_kernel


def _prep1(x, A1, B1):
    def body(x_ref, a_ref, b_ref, o1_ref, o2_ref):
        xv = x_ref[...]
        o1_ref[...] = jnp.dot(
            xv, a_ref[...],
            preferred_element_type=jnp.float32).astype(jnp.bfloat16)
        o2_ref[...] = jnp.dot(xv, b_ref[...], preferred_element_type=jnp.float32)

    return pl.pallas_call(
        body,
        grid=(_N // _BN,),
        in_specs=[
            pl.BlockSpec((_BN, _D), lambda i: (i, 0)),
            pl.BlockSpec((_D, _RS), lambda i: (0, 0)),
            pl.BlockSpec((_D, _RD), lambda i: (0, 0)),
        ],
        out_specs=[
            pl.BlockSpec((_BN, _RS), lambda i: (i, 0)),
            pl.BlockSpec((_BN, _RD), lambda i: (i, 0)),
        ],
        out_shape=[
            jax.ShapeDtypeStruct((_N, _RS), jnp.bfloat16),
            jax.ShapeDtypeStruct((_N, _RD), jnp.float32),
        ],
    )(x, A1, B1)


def _finalize1_prep2(acc1, bysrc1, bydst1, b1, E16, Pinv, A2, B2):
    def body(acc_ref, bs_ref, bd_ref, b1_ref, e_ref, p_ref, a2_ref, b2_ref,
             o1_ref, o2_ref):
        num = acc_ref[0, :, 0:64] + acc_ref[1, :, 0:64]
        den16 = acc_ref[0, :, 64:80] + acc_ref[1, :, 64:80]
        ev = e_ref[...]
        den = jnp.dot(den16, ev, preferred_element_type=jnp.float32)
        bs = jnp.dot(bs_ref[...].astype(jnp.float32), p_ref[...],
                     preferred_element_type=jnp.float32)
        h = bs[:, 0:64]
        a0 = bs[:, 64:80] + bd_ref[...]
        ws16 = jnp.exp(jnp.maximum(a0, 0.2 * a0))
        ws = jnp.dot(ws16, ev, preferred_element_type=jnp.float32)
        o = (num + ws * h) / (den + ws + 1e-16) + b1_ref[...]
        h1 = jnp.where(o > 0, o, jnp.exp(o) - 1.0)
        o1_ref[...] = jnp.dot(
            h1, a2_ref[...],
            preferred_element_type=jnp.float32).astype(jnp.bfloat16)
        o2_ref[...] = jnp.dot(h1, b2_ref[...], preferred_element_type=jnp.float32)

    return pl.pallas_call(
        body,
        grid=(_N // _BN,),
        in_specs=[
            pl.BlockSpec((_NCORES, _BN, _RA), lambda i: (0, i, 0)),
            pl.BlockSpec((_BN, _RS), lambda i: (i, 0)),
            pl.BlockSpec((_BN, _RD), lambda i: (i, 0)),
            pl.BlockSpec((1, 64), lambda i: (0, 0)),
            pl.BlockSpec((16, 64), lambda i: (0, 0)),
            pl.BlockSpec((_RS, _RS), lambda i: (0, 0)),
            pl.BlockSpec((64, _RS), lambda i: (0, 0)),
            pl.BlockSpec((64, _RD), lambda i: (0, 0)),
        ],
        out_specs=[
            pl.BlockSpec((_BN, _RS), lambda i: (i, 0)),
            pl.BlockSpec((_BN, _RD), lambda i: (i, 0)),
        ],
        out_shape=[
            jax.ShapeDtypeStruct((_N, _RS), jnp.bfloat16),
            jax.ShapeDtypeStruct((_N, _RD), jnp.float32),
        ],
    )(acc1, bysrc1, bydst1, b1, E16, Pinv, A2, B2)


def _finalize2(acc2, bysrc2, bydst2, b2, Pinv):
    def body(acc_ref, bs_ref, bd_ref, b2_ref, p_ref, o_ref):
        num = acc_ref[0, :, 0:64] + acc_ref[1, :, 0:64]
        den = acc_ref[0, :, 64:65] + acc_ref[1, :, 64:65]
        bs = jnp.dot(bs_ref[...].astype(jnp.float32), p_ref[...],
                     preferred_element_type=jnp.float32)
        h = bs[:, 0:64]
        a0 = bs[:, 64:65] + bd_ref[:, 0:1]
        ws = jnp.exp(jnp.maximum(a0, 0.2 * a0))
        o = (num + ws * h) / (den + ws + 1e-16) + b2_ref[...]
        m = jnp.max(o, axis=1, keepdims=True)
        z = o - m
        o_ref[...] = z - jnp.log(jnp.sum(jnp.exp(z), axis=1, keepdims=True))

    return pl.pallas_call(
        body,
        grid=(_N // _BN,),
        in_specs=[
            pl.BlockSpec((_NCORES, _BN, _RA), lambda i: (0, i, 0)),
            pl.BlockSpec((_BN, _RS), lambda i: (i, 0)),
            pl.BlockSpec((_BN, _RD), lambda i: (i, 0)),
            pl.BlockSpec((1, 64), lambda i: (0, 0)),
            pl.BlockSpec((_RS, _RS), lambda i: (0, 0)),
        ],
        out_specs=pl.BlockSpec((_BN, 64), lambda i: (i, 0)),
        out_shape=jax.ShapeDtypeStruct((_N, 64), jnp.float32),
    )(acc2, bysrc2, bydst2, b2, Pinv)


def _head_mat(a):
    hh, cc = a.shape
    t = a[:, :, None] * jnp.eye(hh, dtype=a.dtype)[:, None, :]
    return t.reshape(hh * cc, hh)


_edge_pass_cache = {}


def _edge_pass(per_head):
    if per_head not in _edge_pass_cache:
        _edge_pass_cache[per_head] = _make_edge_pass(per_head)
    return _edge_pass_cache[per_head]


def kernel(x, edge_index, W1, a_src1, a_dst1, b1, W2, a_src2, a_dst2, b2):
    n_tiles = _NCORES * _NSUB
    n_chunks = _E // n_tiles // _CHUNK
    src = edge_index[0].reshape(n_tiles, n_chunks, _CHUNK)
    dst = edge_index[1].reshape(n_tiles, n_chunks, _CHUNK)

    f32 = W1.dtype
    zpad = jnp.zeros((64, 8), f32)
    lperm = []
    for g in range(3):
        for k in range(16):
            lperm.extend([32 * g + k, 32 * g + 16 + k])
    lperm = jnp.asarray(lperm, dtype=jnp.int32)
    Pinv = jnp.zeros((_RS, _RS), f32).at[jnp.arange(_RS), lperm].set(1.0)

    A1log = W1 @ jnp.concatenate([jnp.eye(64, dtype=f32), _head_mat(a_src1),
                                  jnp.zeros((64, 24), f32)], axis=1)
    A1 = A1log[:, lperm]
    B1 = W1 @ jnp.concatenate([_head_mat(a_dst1), zpad], axis=1)
    E16 = jnp.concatenate(
        [jnp.repeat(jnp.eye(8, dtype=f32), 8, axis=1), jnp.zeros((8, 64), f32)],
        axis=0)
    s2 = W2 @ a_src2.T
    d2 = W2 @ a_dst2.T
    A2log = jnp.concatenate([W2, jnp.broadcast_to(s2, (64, 16)),
                             jnp.zeros((64, 16), f32)], axis=1)
    A2 = A2log[:, lperm]
    B2 = jnp.broadcast_to(d2, (64, 16))

    lanes = jnp.asarray(
        [[2 * j] * 8 + [2 * j + 1] * 8 for j in range(4)], dtype=jnp.int32)

    bysrc1, bydst1 = _prep1(x, A1, B1)
    acc1 = _edge_pass(True)(src, dst, bysrc1, bydst1, lanes)
    bysrc2, bydst2 = _finalize1_prep2(acc1, bysrc1, bydst1,
                                      b1.reshape(1, 64), E16, Pinv, A2, B2)
    acc2 = _edge_pass(False)(src, dst, bysrc2, bydst2, lanes)
    return _finalize2(acc2, bysrc2, bydst2, b2.reshape(1, 64), Pinv)


# speedup vs baseline: 1.1352x; 1.0134x over previous
import functools

import jax
import jax.numpy as jnp
from jax import lax
from jax.experimental import pallas as pl
from jax.experimental.pallas import tpu as pltpu
from jax.experimental.pallas import tpu_sc as plsc

_N = 10000
_E = 320000
_D = 128

_NCORES = 2
_NSUB = 16
_CHUNK = 80
_BN = 10000
_NBUF = 5

_RS = 96
_RD = 16
_RA = 80


def _make_edge_pass(per_head):
    e_per_sc = _E // _NCORES
    e_per_tile = e_per_sc // _NSUB
    n_chunks = e_per_tile // _CHUNK
    rows_u = 624
    tail0 = rows_u * _NSUB
    tail_rows = _N - tail0
    zrows = 48

    mesh = plsc.VectorSubcoreMesh(core_axis_name="c", subcore_axis_name="s",
                                  num_cores=_NCORES, num_subcores=_NSUB)

    @functools.partial(
        pl.kernel,
        out_type=jax.ShapeDtypeStruct((_NCORES, _N, _RA), jnp.float32),
        mesh=mesh,
        compiler_params=pltpu.CompilerParams(use_tc_tiling_on_sc=False,
                                             needs_layout_passes=False),
        scratch_types=[
            pltpu.VMEM((n_chunks, _CHUNK), jnp.int32),
            pltpu.VMEM((n_chunks, _CHUNK), jnp.int32),
        ] + [pltpu.VMEM((_CHUNK, _RS), jnp.bfloat16)] * _NBUF
          + [pltpu.VMEM((_CHUNK, _RD), jnp.float32)] * _NBUF
          + [pltpu.VMEM((_CHUNK, _RA), jnp.float32)] * _NBUF
          + [pltpu.VMEM((4, 16), jnp.int32),
             pltpu.VMEM_SHARED((_N, _RA), jnp.float32)]
          + [pltpu.SemaphoreType.DMA] * (2 * _NBUF),
    )
    def edge_kernel(src_h, dst_h, bysrc_h, bydst_h, lanes_h, out_h,
                    src_i, dst_i, *rest):
        rsrc = rest[0:_NBUF]
        rdst = rest[_NBUF:2 * _NBUF]
        msg = rest[2 * _NBUF:3 * _NBUF]
        lanes_v, acc = rest[3 * _NBUF:3 * _NBUF + 2]
        gsem = rest[3 * _NBUF + 2:4 * _NBUF + 2]
        ssem = rest[4 * _NBUF + 2:5 * _NBUF + 2]
        cid = lax.axis_index("c")
        sid = lax.axis_index("s")
        tid = cid * _NSUB + sid
        row0 = sid * rows_u

        idx_copies = [
            pltpu.async_copy(src_h.at[tid], src_i, gsem[0]),
            pltpu.async_copy(dst_h.at[tid], dst_i, gsem[1]),
        ]
        pltpu.sync_copy(lanes_h, lanes_v)

        zero16 = jnp.zeros((16,), jnp.float32)
        zbuf = msg[0]

        def zero_row(i, _):
            for j in range(_RA // 16):
                zbuf[i, pl.ds(16 * j, 16)] = zero16
            return 0

        def issue(c, b):
            pltpu.async_copy(bysrc_h.at[src_i.at[c]], rsrc[b], gsem[b])
            pltpu.async_copy(bydst_h.at[dst_i.at[c]], rdst[b], gsem[b])

        def wait_gathers(b):
            pltpu.make_async_copy(bysrc_h.at[src_i.at[0]], rsrc[b],
                                  gsem[b]).wait()
            pltpu.make_async_copy(bydst_h.at[dst_i.at[0]], rdst[b],
                                  gsem[b]).wait()

        def wait_scatter(b):
            pltpu.make_async_copy(msg[b], acc.at[dst_i.at[0]],
                                  ssem[b]).wait()

        unroll = 8

        def edge_body(b):
            lanes = [lanes_v[j, :] for j in range(4)]

            def one_edge(ei):
                s16, _ = plsc.unpack(rsrc[b][ei, pl.ds(64, 32)],
                                     format=plsc.PackFormat.INTERLEAVED,
                                     preferred_element_type=jnp.float32)
                a = s16 + rdst[b][ei, pl.ds(0, 16)]
                w = jnp.exp(jnp.maximum(a, 0.2 * a))
                msg[b][ei, pl.ds(64, 16)] = w
                for g in range(2):
                    h0, h1 = plsc.unpack(rsrc[b][ei, pl.ds(32 * g, 32)],
                                         format=plsc.PackFormat.INTERLEAVED,
                                         preferred_element_type=jnp.float32)
                    for jj, hv in ((2 * g, h0), (2 * g + 1, h1)):
                        if per_head:
                            wj = w.at[lanes[jj]].get(
                                mode="promise_in_bounds")
                        else:
                            wj = w
                        msg[b][ei, pl.ds(16 * jj, 16)] = hv * wj

            plsc.parallel_loop(0, _CHUNK, 1, unroll=unroll)(one_edge)

        lax.fori_loop(0, _CHUNK, zero_row, 0)
        for c in idx_copies:
            c.wait()
        for b in range(_NBUF):
            issue(b, b)
        for k in range(rows_u // _CHUNK):
            pltpu.sync_copy(zbuf, acc.at[pl.ds(row0 + k * _CHUNK, _CHUNK)])
        rem = rows_u - (rows_u // _CHUNK) * _CHUNK
        pltpu.sync_copy(zbuf.at[pl.ds(0, rem)],
                        acc.at[pl.ds(row0 + rows_u - rem, rem)])

        @pl.when(sid == _NSUB - 1)
        def _():
            pltpu.sync_copy(zbuf.at[pl.ds(0, tail_rows)],
                            acc.at[pl.ds(tail0, tail_rows)])

        plsc.subcore_barrier()

        def stage(c, b):
            wait_gathers(b)

            @pl.when(c >= _NBUF)
            def _():
                wait_scatter(b)
            edge_body(b)
            pltpu.async_copy(msg[b], acc.at[dst_i.at[c]], ssem[b], add=True)

            @pl.when(c + _NBUF < n_chunks)
            def _():
                issue(c + _NBUF, b)

        def ring_body(k, _):
            for u in range(_NBUF):
                stage(_NBUF * k + u, u)
            return 0

        lax.fori_loop(0, n_chunks // _NBUF, ring_body, 0)
        for b in range(_NBUF):
            wait_scatter(b)
        plsc.subcore_barrier()
        pltpu.sync_copy(acc.at[pl.ds(row0, rows_u)],
                        out_h.at[cid, pl.ds(row0, rows_u)])

        @pl.when(sid == _NSUB - 1)
        def _():
            pltpu.sync_copy(acc.at[pl.ds(tail0, tail_rows)],
                            out_h.at[cid, pl.ds(tail0, tail_rows)])

    return edge_kernel


def _prep1(x, A1, B1):
    def body(x_ref, a_ref, b_ref, o1_ref, o2_ref):
        xv = x_ref[...]
        o1_ref[...] = jnp.dot(
            xv, a_ref[...],
            preferred_element_type=jnp.float32).astype(jnp.bfloat16)
        o2_ref[...] = jnp.dot(xv, b_ref[...], preferred_element_type=jnp.float32)

    return pl.pallas_call(
        body,
        grid=(_N // _BN,),
        in_specs=[
            pl.BlockSpec((_BN, _D), lambda i: (i, 0)),
            pl.BlockSpec((_D, _RS), lambda i: (0, 0)),
            pl.BlockSpec((_D, _RD), lambda i: (0, 0)),
        ],
        out_specs=[
            pl.BlockSpec((_BN, _RS), lambda i: (i, 0)),
            pl.BlockSpec((_BN, _RD), lambda i: (i, 0)),
        ],
        out_shape=[
            jax.ShapeDtypeStruct((_N, _RS), jnp.bfloat16),
            jax.ShapeDtypeStruct((_N, _RD), jnp.float32),
        ],
    )(x, A1, B1)


def _finalize1_prep2(acc1, bysrc1, bydst1, b1, E16, Pinv, A2, B2):
    def body(acc_ref, bs_ref, bd_ref, b1_ref, e_ref, p_ref, a2_ref, b2_ref,
             o1_ref, o2_ref):
        num = acc_ref[0, :, 0:64] + acc_ref[1, :, 0:64]
        den16 = acc_ref[0, :, 64:80] + acc_ref[1, :, 64:80]
        ev = e_ref[...]
        den = jnp.dot(den16, ev, preferred_element_type=jnp.float32)
        bs = jnp.dot(bs_ref[...].astype(jnp.float32), p_ref[...],
                     preferred_element_type=jnp.float32)
        h = bs[:, 0:64]
        a0 = bs[:, 64:80] + bd_ref[...]
        ws16 = jnp.exp(jnp.maximum(a0, 0.2 * a0))
        ws = jnp.dot(ws16, ev, preferred_element_type=jnp.float32)
        o = (num + ws * h) / (den + ws + 1e-16) + b1_ref[...]
        h1 = jnp.where(o > 0, o, jnp.exp(o) - 1.0)
        o1_ref[...] = jnp.dot(
            h1, a2_ref[...],
            preferred_element_type=jnp.float32).astype(jnp.bfloat16)
        o2_ref[...] = jnp.dot(h1, b2_ref[...], preferred_element_type=jnp.float32)

    return pl.pallas_call(
        body,
        grid=(_N // _BN,),
        in_specs=[
            pl.BlockSpec((_NCORES, _BN, _RA), lambda i: (0, i, 0)),
            pl.BlockSpec((_BN, _RS), lambda i: (i, 0)),
            pl.BlockSpec((_BN, _RD), lambda i: (i, 0)),
            pl.BlockSpec((1, 64), lambda i: (0, 0)),
            pl.BlockSpec((16, 64), lambda i: (0, 0)),
            pl.BlockSpec((_RS, _RS), lambda i: (0, 0)),
            pl.BlockSpec((64, _RS), lambda i: (0, 0)),
            pl.BlockSpec((64, _RD), lambda i: (0, 0)),
        ],
        out_specs=[
            pl.BlockSpec((_BN, _RS), lambda i: (i, 0)),
            pl.BlockSpec((_BN, _RD), lambda i: (i, 0)),
        ],
        out_shape=[
            jax.ShapeDtypeStruct((_N, _RS), jnp.bfloat16),
            jax.ShapeDtypeStruct((_N, _RD), jnp.float32),
        ],
    )(acc1, bysrc1, bydst1, b1, E16, Pinv, A2, B2)


def _finalize2(acc2, bysrc2, bydst2, b2, Pinv):
    def body(acc_ref, bs_ref, bd_ref, b2_ref, p_ref, o_ref):
        num = acc_ref[0, :, 0:64] + acc_ref[1, :, 0:64]
        den = acc_ref[0, :, 64:65] + acc_ref[1, :, 64:65]
        bs = jnp.dot(bs_ref[...].astype(jnp.float32), p_ref[...],
                     preferred_element_type=jnp.float32)
        h = bs[:, 0:64]
        a0 = bs[:, 64:65] + bd_ref[:, 0:1]
        ws = jnp.exp(jnp.maximum(a0, 0.2 * a0))
        o = (num + ws * h) / (den + ws + 1e-16) + b2_ref[...]
        m = jnp.max(o, axis=1, keepdims=True)
        z = o - m
        o_ref[...] = z - jnp.log(jnp.sum(jnp.exp(z), axis=1, keepdims=True))

    return pl.pallas_call(
        body,
        grid=(_N // _BN,),
        in_specs=[
            pl.BlockSpec((_NCORES, _BN, _RA), lambda i: (0, i, 0)),
            pl.BlockSpec((_BN, _RS), lambda i: (i, 0)),
            pl.BlockSpec((_BN, _RD), lambda i: (i, 0)),
            pl.BlockSpec((1, 64), lambda i: (0, 0)),
            pl.BlockSpec((_RS, _RS), lambda i: (0, 0)),
        ],
        out_specs=pl.BlockSpec((_BN, 64), lambda i: (i, 0)),
        out_shape=jax.ShapeDtypeStruct((_N, 64), jnp.float32),
    )(acc2, bysrc2, bydst2, b2, Pinv)


def _head_mat(a):
    hh, cc = a.shape
    t = a[:, :, None] * jnp.eye(hh, dtype=a.dtype)[:, None, :]
    return t.reshape(hh * cc, hh)


_edge_pass_cache = {}


def _edge_pass(per_head):
    if per_head not in _edge_pass_cache:
        _edge_pass_cache[per_head] = _make_edge_pass(per_head)
    return _edge_pass_cache[per_head]


def kernel(x, edge_index, W1, a_src1, a_dst1, b1, W2, a_src2, a_dst2, b2):
    n_tiles = _NCORES * _NSUB
    n_chunks = _E // n_tiles // _CHUNK
    src = edge_index[0].reshape(n_tiles, n_chunks, _CHUNK)
    dst = edge_index[1].reshape(n_tiles, n_chunks, _CHUNK)

    f32 = W1.dtype
    zpad = jnp.zeros((64, 8), f32)
    lperm = []
    for g in range(3):
        for k in range(16):
            lperm.extend([32 * g + k, 32 * g + 16 + k])
    lperm = jnp.asarray(lperm, dtype=jnp.int32)
    Pinv = jnp.zeros((_RS, _RS), f32).at[jnp.arange(_RS), lperm].set(1.0)

    A1log = W1 @ jnp.concatenate([jnp.eye(64, dtype=f32), _head_mat(a_src1),
                                  jnp.zeros((64, 24), f32)], axis=1)
    A1 = A1log[:, lperm]
    B1 = W1 @ jnp.concatenate([_head_mat(a_dst1), zpad], axis=1)
    E16 = jnp.concatenate(
        [jnp.repeat(jnp.eye(8, dtype=f32), 8, axis=1), jnp.zeros((8, 64), f32)],
        axis=0)
    s2 = W2 @ a_src2.T
    d2 = W2 @ a_dst2.T
    A2log = jnp.concatenate([W2, jnp.broadcast_to(s2, (64, 16)),
                             jnp.zeros((64, 16), f32)], axis=1)
    A2 = A2log[:, lperm]
    B2 = jnp.broadcast_to(d2, (64, 16))

    lanes = jnp.asarray(
        [[2 * j] * 8 + [2 * j + 1] * 8 for j in range(4)], dtype=jnp.int32)

    bysrc1, bydst1 = _prep1(x, A1, B1)
    acc1 = _edge_pass(True)(src, dst, bysrc1, bydst1, lanes)
    bysrc2, bydst2 = _finalize1_prep2(acc1, bysrc1, bydst1,
                                      b1.reshape(1, 64), E16, Pinv, A2, B2)
    acc2 = _edge_pass(False)(src, dst, bysrc2, bydst2, lanes)
    return _finalize2(acc2, bysrc2, bydst2, b2.reshape(1, 64), Pinv)


# speedup vs baseline: 1.1355x; 1.0003x over previous
import functools

import jax
import jax.numpy as jnp
from jax import lax
from jax.experimental import pallas as pl
from jax.experimental.pallas import tpu as pltpu
from jax.experimental.pallas import tpu_sc as plsc

_N = 10000
_E = 320000
_D = 128

_NCORES = 2
_NSUB = 16
_CHUNK = 80
_BN = 10000
_NBUF = 5

_RS = 96
_RD = 16
_RA = 80


def _make_edge_pass(per_head):
    e_per_sc = _E // _NCORES
    e_per_tile = e_per_sc // _NSUB
    n_chunks = e_per_tile // _CHUNK
    rows_u = 624
    tail0 = rows_u * _NSUB
    tail_rows = _N - tail0

    mesh = plsc.VectorSubcoreMesh(core_axis_name="c", subcore_axis_name="s",
                                  num_cores=_NCORES, num_subcores=_NSUB)

    @functools.partial(
        pl.kernel,
        out_type=jax.ShapeDtypeStruct((_NCORES, _N, _RA), jnp.float32),
        mesh=mesh,
        compiler_params=pltpu.CompilerParams(use_tc_tiling_on_sc=False,
                                             needs_layout_passes=False),
        scratch_types=[
            pltpu.VMEM((n_chunks, _CHUNK), jnp.int32),
            pltpu.VMEM((n_chunks, _CHUNK), jnp.int32),
        ] + [pltpu.VMEM((_CHUNK, _RS), jnp.bfloat16)] * _NBUF
          + [pltpu.VMEM((_CHUNK, _RD), jnp.float32)] * _NBUF
          + [pltpu.VMEM((_CHUNK, _RA), jnp.float32)] * _NBUF
          + [pltpu.VMEM((4, 16), jnp.int32),
             pltpu.VMEM_SHARED((_N, _RA), jnp.float32)]
          + [pltpu.SemaphoreType.DMA] * (2 * _NBUF),
    )
    def edge_kernel(src_h, dst_h, bysrc_h, bydst_h, lanes_h, out_h,
                    src_i, dst_i, *rest):
        rsrc = rest[0:_NBUF]
        rdst = rest[_NBUF:2 * _NBUF]
        msg = rest[2 * _NBUF:3 * _NBUF]
        lanes_v, acc = rest[3 * _NBUF:3 * _NBUF + 2]
        gsem = rest[3 * _NBUF + 2:4 * _NBUF + 2]
        ssem = rest[4 * _NBUF + 2:5 * _NBUF + 2]
        cid = lax.axis_index("c")
        sid = lax.axis_index("s")
        tid = cid * _NSUB + sid
        row0 = sid * rows_u

        idx_copies = [
            pltpu.async_copy(src_h.at[tid], src_i, gsem[0]),
            pltpu.async_copy(dst_h.at[tid], dst_i, gsem[1]),
        ]
        pltpu.sync_copy(lanes_h, lanes_v)

        zero16 = jnp.zeros((16,), jnp.float32)
        zbuf = msg[0]

        def zero_row(i, _):
            for j in range(_RA // 16):
                zbuf[i, pl.ds(16 * j, 16)] = zero16
            return 0

        def issue(c, b):
            pltpu.async_copy(bysrc_h.at[src_i.at[c]], rsrc[b], gsem[b])
            pltpu.async_copy(bydst_h.at[dst_i.at[c]], rdst[b], gsem[b])

        def wait_gathers(b):
            pltpu.make_async_copy(bysrc_h.at[src_i.at[0]], rsrc[b],
                                  gsem[b]).wait()
            pltpu.make_async_copy(bydst_h.at[dst_i.at[0]], rdst[b],
                                  gsem[b]).wait()

        def wait_scatter(b):
            pltpu.make_async_copy(msg[b], acc.at[dst_i.at[0]],
                                  ssem[b]).wait()

        unroll = 8

        def edge_body(b):
            lanes = [lanes_v[j, :] for j in range(4)]

            def one_edge(ei):
                s16, _ = plsc.unpack(rsrc[b][ei, pl.ds(64, 32)],
                                     format=plsc.PackFormat.INTERLEAVED,
                                     preferred_element_type=jnp.float32)
                a = s16 + rdst[b][ei, pl.ds(0, 16)]
                w = jnp.exp(jnp.maximum(a, 0.2 * a))
                msg[b][ei, pl.ds(64, 16)] = w
                for g in range(2):
                    h0, h1 = plsc.unpack(rsrc[b][ei, pl.ds(32 * g, 32)],
                                         format=plsc.PackFormat.INTERLEAVED,
                                         preferred_element_type=jnp.float32)
                    for jj, hv in ((2 * g, h0), (2 * g + 1, h1)):
                        if per_head:
                            wj = w.at[lanes[jj]].get(
                                mode="promise_in_bounds")
                        else:
                            wj = w
                        msg[b][ei, pl.ds(16 * jj, 16)] = hv * wj

            plsc.parallel_loop(0, _CHUNK, 1, unroll=unroll)(one_edge)

        lax.fori_loop(0, _CHUNK, zero_row, 0)
        for c in idx_copies:
            c.wait()
        for b in range(_NBUF):
            issue(b, b)
        for k in range(rows_u // _CHUNK):
            pltpu.sync_copy(zbuf, acc.at[pl.ds(row0 + k * _CHUNK, _CHUNK)])
        rem = rows_u - (rows_u // _CHUNK) * _CHUNK
        pltpu.sync_copy(zbuf.at[pl.ds(0, rem)],
                        acc.at[pl.ds(row0 + rows_u - rem, rem)])

        @pl.when(sid == _NSUB - 1)
        def _():
            pltpu.sync_copy(zbuf.at[pl.ds(0, tail_rows)],
                            acc.at[pl.ds(tail0, tail_rows)])

        plsc.subcore_barrier()

        def stage(c, b):
            wait_gathers(b)

            @pl.when(c >= _NBUF)
            def _():
                wait_scatter(b)
            edge_body(b)
            pltpu.async_copy(msg[b], acc.at[dst_i.at[c]], ssem[b], add=True)

            @pl.when(c + _NBUF < n_chunks)
            def _():
                issue(c + _NBUF, b)

        def ring_body(k, _):
            for u in range(_NBUF):
                stage(_NBUF * k + u, u)
            return 0

        lax.fori_loop(0, n_chunks // _NBUF, ring_body, 0)
        for b in range(_NBUF):
            wait_scatter(b)
        plsc.subcore_barrier()
        pltpu.sync_copy(acc.at[pl.ds(row0, rows_u)],
                        out_h.at[cid, pl.ds(row0, rows_u)])

        @pl.when(sid == _NSUB - 1)
        def _():
            pltpu.sync_copy(acc.at[pl.ds(tail0, tail_rows)],
                            out_h.at[cid, pl.ds(tail0, tail_rows)])

    return edge_kernel


def _prep1(x, A1, B1):
    def body(x_ref, a_ref, b_ref, o1_ref, o2_ref):
        xv = x_ref[...]
        o1_ref[...] = jnp.dot(
            xv, a_ref[...],
            preferred_element_type=jnp.float32).astype(jnp.bfloat16)
        o2_ref[...] = jnp.dot(xv, b_ref[...], preferred_element_type=jnp.float32)

    return pl.pallas_call(
        body,
        grid=(_N // _BN,),
        in_specs=[
            pl.BlockSpec((_BN, _D), lambda i: (i, 0)),
            pl.BlockSpec((_D, _RS), lambda i: (0, 0)),
            pl.BlockSpec((_D, _RD), lambda i: (0, 0)),
        ],
        out_specs=[
            pl.BlockSpec((_BN, _RS), lambda i: (i, 0)),
            pl.BlockSpec((_BN, _RD), lambda i: (i, 0)),
        ],
        out_shape=[
            jax.ShapeDtypeStruct((_N, _RS), jnp.bfloat16),
            jax.ShapeDtypeStruct((_N, _RD), jnp.float32),
        ],
    )(x, A1, B1)


def _finalize1_prep2(acc1, bysrc1, bydst1, b1, E16, Pinv, A2, B2):
    def body(acc_ref, bs_ref, bd_ref, b1_ref, e_ref, p_ref, a2_ref, b2_ref,
             o1_ref, o2_ref):
        num = acc_ref[0, :, 0:64] + acc_ref[1, :, 0:64]
        den16 = acc_ref[0, :, 64:80] + acc_ref[1, :, 64:80]
        ev = e_ref[...]
        den = jnp.dot(den16, ev, preferred_element_type=jnp.float32)
        bs = jnp.dot(bs_ref[...].astype(jnp.float32), p_ref[...],
                     preferred_element_type=jnp.float32)
        h = bs[:, 0:64]
        a0 = bs[:, 64:80] + bd_ref[...]
        ws16 = jnp.exp(jnp.maximum(a0, 0.2 * a0))
        ws = jnp.dot(ws16, ev, preferred_element_type=jnp.float32)
        o = (num + ws * h) / (den + ws + 1e-16) + b1_ref[...]
        h1 = jnp.where(o > 0, o, jnp.exp(o) - 1.0)
        o1_ref[...] = jnp.dot(
            h1, a2_ref[...],
            preferred_element_type=jnp.float32).astype(jnp.bfloat16)
        o2_ref[...] = jnp.dot(h1, b2_ref[...], preferred_element_type=jnp.float32)

    return pl.pallas_call(
        body,
        grid=(_N // _BN,),
        in_specs=[
            pl.BlockSpec((_NCORES, _BN, _RA), lambda i: (0, i, 0)),
            pl.BlockSpec((_BN, _RS), lambda i: (i, 0)),
            pl.BlockSpec((_BN, _RD), lambda i: (i, 0)),
            pl.BlockSpec((1, 64), lambda i: (0, 0)),
            pl.BlockSpec((16, 64), lambda i: (0, 0)),
            pl.BlockSpec((_RS, _RS), lambda i: (0, 0)),
            pl.BlockSpec((64, _RS), lambda i: (0, 0)),
            pl.BlockSpec((64, _RD), lambda i: (0, 0)),
        ],
        out_specs=[
            pl.BlockSpec((_BN, _RS), lambda i: (i, 0)),
            pl.BlockSpec((_BN, _RD), lambda i: (i, 0)),
        ],
        out_shape=[
            jax.ShapeDtypeStruct((_N, _RS), jnp.bfloat16),
            jax.ShapeDtypeStruct((_N, _RD), jnp.float32),
        ],
    )(acc1, bysrc1, bydst1, b1, E16, Pinv, A2, B2)


def _finalize2(acc2, bysrc2, bydst2, b2, Pinv):
    def body(acc_ref, bs_ref, bd_ref, b2_ref, p_ref, o_ref):
        num = acc_ref[0, :, 0:64] + acc_ref[1, :, 0:64]
        den = acc_ref[0, :, 64:65] + acc_ref[1, :, 64:65]
        bs = jnp.dot(bs_ref[...].astype(jnp.float32), p_ref[...],
                     preferred_element_type=jnp.float32)
        h = bs[:, 0:64]
        a0 = bs[:, 64:65] + bd_ref[:, 0:1]
        ws = jnp.exp(jnp.maximum(a0, 0.2 * a0))
        o = (num + ws * h) / (den + ws + 1e-16) + b2_ref[...]
        m = jnp.max(o, axis=1, keepdims=True)
        z = o - m
        o_ref[...] = z - jnp.log(jnp.sum(jnp.exp(z), axis=1, keepdims=True))

    return pl.pallas_call(
        body,
        grid=(_N // _BN,),
        in_specs=[
            pl.BlockSpec((_NCORES, _BN, _RA), lambda i: (0, i, 0)),
            pl.BlockSpec((_BN, _RS), lambda i: (i, 0)),
            pl.BlockSpec((_BN, _RD), lambda i: (i, 0)),
            pl.BlockSpec((1, 64), lambda i: (0, 0)),
            pl.BlockSpec((_RS, _RS), lambda i: (0, 0)),
        ],
        out_specs=pl.BlockSpec((_BN, 64), lambda i: (i, 0)),
        out_shape=jax.ShapeDtypeStruct((_N, 64), jnp.float32),
    )(acc2, bysrc2, bydst2, b2, Pinv)


def _head_mat(a):
    hh, cc = a.shape
    t = a[:, :, None] * jnp.eye(hh, dtype=a.dtype)[:, None, :]
    return t.reshape(hh * cc, hh)


_edge_pass_cache = {}


def _edge_pass(per_head):
    if per_head not in _edge_pass_cache:
        _edge_pass_cache[per_head] = _make_edge_pass(per_head)
    return _edge_pass_cache[per_head]


def kernel(x, edge_index, W1, a_src1, a_dst1, b1, W2, a_src2, a_dst2, b2):
    n_tiles = _NCORES * _NSUB
    n_chunks = _E // n_tiles // _CHUNK
    src = edge_index[0].reshape(n_tiles, n_chunks, _CHUNK)
    dst = edge_index[1].reshape(n_tiles, n_chunks, _CHUNK)

    f32 = W1.dtype
    zpad = jnp.zeros((64, 8), f32)
    lperm = []
    for g in range(3):
        for k in range(16):
            lperm.extend([32 * g + k, 32 * g + 16 + k])
    lperm = jnp.asarray(lperm, dtype=jnp.int32)
    Pinv = jnp.zeros((_RS, _RS), f32).at[jnp.arange(_RS), lperm].set(1.0)

    A1log = W1 @ jnp.concatenate([jnp.eye(64, dtype=f32), _head_mat(a_src1),
                                  jnp.zeros((64, 24), f32)], axis=1)
    A1 = A1log[:, lperm]
    B1 = W1 @ jnp.concatenate([_head_mat(a_dst1), zpad], axis=1)
    E16 = jnp.concatenate(
        [jnp.repeat(jnp.eye(8, dtype=f32), 8, axis=1), jnp.zeros((8, 64), f32)],
        axis=0)
    s2 = W2 @ a_src2.T
    d2 = W2 @ a_dst2.T
    A2log = jnp.concatenate([W2, jnp.broadcast_to(s2, (64, 16)),
                             jnp.zeros((64, 16), f32)], axis=1)
    A2 = A2log[:, lperm]
    B2 = jnp.broadcast_to(d2, (64, 16))

    lanes = jnp.asarray(
        [[2 * j] * 8 + [2 * j + 1] * 8 for j in range(4)], dtype=jnp.int32)

    bysrc1, bydst1 = _prep1(x, A1, B1)
    acc1 = _edge_pass(True)(src, dst, bysrc1, bydst1, lanes)
    bysrc2, bydst2 = _finalize1_prep2(acc1, bysrc1, bydst1,
                                      b1.reshape(1, 64), E16, Pinv, A2, B2)
    acc2 = _edge_pass(False)(src, dst, bysrc2, bydst2, lanes)
    return _finalize2(acc2, bysrc2, bydst2, b2.reshape(1, 64), Pinv)
